# baseline (speedup 1.0000x reference)
import functools

import jax
import jax.numpy as jnp
from jax import lax
from jax.experimental import pallas as pl
from jax.experimental.pallas import tpu as pltpu
from jax.experimental.pallas import tpu_sc as plsc

NC = 2
NS = 16
NW = NC * NS
LANES = 16
CHUNK = 128
NG = 2
NO = 2


def _make_sc_kernel(n_rows, max_len, emb_dim):
    assert n_rows % (NW * CHUNK) == 0
    assert emb_dim % LANES == 0 and 128 % emb_dim == 0
    per_row = 128 // emb_dim
    rows_per_w = n_rows // NW
    chunks_per_w = rows_per_w // CHUNK
    assert chunks_per_w % NG == 0 and chunks_per_w % NO == 0
    n_slices = emb_dim // LANES
    opair_per_chunk = CHUNK // per_row
    pe_pairs = max_len // per_row
    mesh = plsc.VectorSubcoreMesh(core_axis_name="c", subcore_axis_name="s")

    @functools.partial(
        pl.kernel,
        mesh=mesh,
        out_type=jax.ShapeDtypeStruct((n_rows // per_row, 128), jnp.float32),
        scratch_types=[
            pltpu.VMEM((chunks_per_w, CHUNK), jnp.int32),
            [pltpu.VMEM((CHUNK,), jnp.int32) for _ in range(NG)],
            [pltpu.VMEM((CHUNK, 128), jnp.float32) for _ in range(NG)],
            [pltpu.VMEM((opair_per_chunk, 128), jnp.float32) for _ in range(NO)],
            pltpu.VMEM((2 * pe_pairs, 128), jnp.float32),
            [pltpu.SemaphoreType.DMA for _ in range(NG)],
            [pltpu.SemaphoreType.DMA for _ in range(NO)],
        ],
        compiler_params=pltpu.CompilerParams(use_tc_tiling_on_sc=True),
    )
    def k(x_hbm, table_hbm, pe_hbm, out_hbm, idx_v, hidx, gbufs, obufs, pe_v,
          gsems, ssems):
        wid = lax.axis_index("s") * NC + lax.axis_index("c")
        obase = wid * (rows_per_w // per_row)
        pltpu.sync_copy(x_hbm.at[pl.ds(wid * chunks_per_w, chunks_per_w)], idx_v)
        pltpu.sync_copy(pe_hbm, pe_v.at[pl.ds(0, pe_pairs)])
        pltpu.sync_copy(pe_hbm, pe_v.at[pl.ds(pe_pairs, pe_pairs)])

        def start_gather(c, g):
            @plsc.parallel_loop(0, CHUNK // LANES, step=1, unroll=2)
            def _(i):
                sl = pl.ds(i * LANES, LANES)
                hidx[g][sl] = jax.lax.shift_right_logical(idx_v[c, sl], 1)

            pltpu.async_copy(table_hbm.at[hidx[g]], gbufs[g], gsems[g])

        def out_slice(c):
            return out_hbm.at[pl.ds(obase + c * opair_per_chunk, opair_per_chunk)]

        for g in range(NG - 1):
            start_gather(g, g)

        def chunk_step(c, g, o):
            @pl.when(c + NG - 1 < chunks_per_w)
            def _():
                start_gather(c + NG - 1, (g + NG - 1) % NG)

            gbuf, obuf = gbufs[g], obufs[o]
            pltpu.make_async_copy(table_hbm.at[hidx[g]], gbuf, gsems[g]).wait()

            @pl.when(c >= NO)
            def _():
                pltpu.make_async_copy(obuf, out_slice(c - NO), ssems[o]).wait()

            pstart = lax.rem(c * opair_per_chunk, pe_pairs)

            @plsc.parallel_loop(0, CHUNK // LANES, step=1, unroll=2)
            def _(rg):
                r0 = rg * LANES
                vec = idx_v[c, pl.ds(r0, LANES)]
                for l in range(LANES):
                    rp = jax.lax.shift_right_logical(r0, 1) + (l >> 1)
                    half = (l & 1) * emb_dim
                    src_half = (vec[l] & (per_row - 1)) * emb_dim
                    p = pstart + rp
                    for j in range(n_slices):
                        obuf[rp, pl.ds(half + j * LANES, LANES)] = (
                            gbuf[r0 + l, pl.ds(src_half + j * LANES, LANES)]
                            + pe_v[p, pl.ds(half + j * LANES, LANES)]
                        )

            pltpu.async_copy(obuf, out_slice(c), ssems[o])

        def group_body(q, carry):
            c0 = q * NG
            for b in range(NG):
                chunk_step(c0 + b, b, b % NO)
            return carry

        lax.fori_loop(0, chunks_per_w // NG, group_body, 0)

        for b in range(NO):
            c = chunks_per_w - NO + b
            pltpu.make_async_copy(obufs[c % NO], out_slice(c), ssems[c % NO]).wait()

    return k


def kernel(x, table, pe):
    batch, max_len = x.shape
    n_vocab, emb_dim = table.shape
    n_rows = batch * max_len
    per_row = 128 // emb_dim
    x_flat = x.reshape(n_rows // CHUNK, CHUNK).astype(jnp.int32)
    table_p = table.reshape(n_vocab // per_row, 128)
    pe_p = pe.reshape(max_len // per_row, 128).astype(jnp.float32)
    k = _make_sc_kernel(n_rows, max_len, emb_dim)
    out = k(x_flat, table_p, pe_p)
    return out.reshape(batch, max_len, emb_dim)
